# Initial kernel scaffold; baseline (speedup 1.0000x reference)
import jax
import jax.numpy as jnp
from jax.experimental import pallas as pl


def kernel(x, edge_index, W_l0, b_l0, W_r0, gamma0, beta0, W_l1, b_l1, W_r1):
    raise NotImplementedError("write your pallas kernel here")


# speedup vs baseline: 5.4135x; 5.4135x over previous
import functools

import jax
import jax.numpy as jnp
from jax import lax
from jax.experimental import pallas as pl
from jax.experimental.pallas import tpu as pltpu
from jax.experimental.pallas import tpu_sc as plsc

N = 10000
NPAD = 10240
E = 320000
NFEAT = 128
NHID = 128
NCLASS = 47
CPAD = 48
BN_EPS = 1e-5

NC = 2
NS = 16
NW = NC * NS
EPW = E // NW
K = 80
NCHUNK = EPW // K
RPT = NPAD // NS
ZB = 40


def _make_segsum(D, with_degree):
  mesh = plsc.VectorSubcoreMesh(core_axis_name="c", subcore_axis_name="s")
  out_type = [jax.ShapeDtypeStruct((NC, NPAD, D), jnp.float32)]
  scratch = [
      pltpu.VMEM((K,), jnp.int32),
      pltpu.VMEM((K,), jnp.int32),
      pltpu.VMEM((K, D), jnp.float32),
      pltpu.VMEM((ZB, D), jnp.float32),
      pltpu.VMEM_SHARED((NPAD, D), jnp.float32),
      pltpu.SemaphoreType.DMA,
  ]
  if with_degree:
    out_type.append(jax.ShapeDtypeStruct((NC, NPAD, 16), jnp.float32))
    scratch += [
        pltpu.VMEM((K, 16), jnp.float32),
        pltpu.VMEM((ZB, 16), jnp.float32),
        pltpu.VMEM_SHARED((NPAD, 16), jnp.float32),
    ]

  def body(feat, src, dst, *refs):
    if with_degree:
      (out, deg_out, idx_s, idx_d, rows, zbuf, acc, sem,
       ones, dzbuf, dacc) = refs
    else:
      out, idx_s, idx_d, rows, zbuf, acc, sem = refs
    c = lax.axis_index("c")
    s = lax.axis_index("s")

    zv = jnp.zeros((16,), jnp.float32)
    for r in range(ZB):
      for j in range(D // 16):
        zbuf[r, pl.ds(j * 16, 16)] = zv
    if with_degree:
      ov = jnp.ones((16,), jnp.float32)
      for r in range(K):
        ones[r] = ov
      for r in range(ZB):
        dzbuf[r] = zv

    r0 = s * RPT

    def zero_loop(i, _):
      pltpu.sync_copy(zbuf, acc.at[pl.ds(r0 + i * ZB, ZB)])
      if with_degree:
        pltpu.sync_copy(dzbuf, dacc.at[pl.ds(r0 + i * ZB, ZB)])
      return 0

    lax.fori_loop(0, RPT // ZB, zero_loop, 0)
    plsc.subcore_barrier()

    base0 = (c * NS + s) * EPW

    def edge_loop(i, _):
      base = base0 + i * K
      pltpu.sync_copy(src.at[pl.ds(base, K)], idx_s)
      pltpu.sync_copy(dst.at[pl.ds(base, K)], idx_d)
      pltpu.async_copy(feat.at[idx_s], rows, sem).wait()
      pltpu.sync_copy(rows, acc.at[idx_d], add=True)
      if with_degree:
        pltpu.sync_copy(ones, dacc.at[idx_d], add=True)
      return 0

    lax.fori_loop(0, NCHUNK, edge_loop, 0)
    plsc.subcore_barrier()

    pltpu.sync_copy(acc.at[pl.ds(r0, RPT)], out.at[c, pl.ds(r0, RPT)])
    if with_degree:
      pltpu.sync_copy(dacc.at[pl.ds(r0, RPT)], deg_out.at[c, pl.ds(r0, RPT)])

  return pl.kernel(body, out_type=tuple(out_type), mesh=mesh,
                   scratch_types=tuple(scratch),
                   compiler_params=pltpu.CompilerParams(
                       use_tc_tiling_on_sc=False))


_segsum_feat = _make_segsum(NFEAT, with_degree=True)
_segsum_cls = _make_segsum(CPAD, with_degree=False)

BR = 1024


def _dense0_body(s0p, degp, x, wl0, bl0, wr0, scale, shift, wl1, h_out, q_out):
  ssum = s0p[0] + s0p[1]
  deg = degp[0, :, 0:1] + degp[1, :, 0:1]
  mean = ssum / jnp.maximum(deg, 1.0)
  z = (jax.lax.dot(mean, wl0[...], preferred_element_type=jnp.float32)
       + bl0[...]
       + jax.lax.dot(x[...], wr0[...], preferred_element_type=jnp.float32))
  h = jnp.maximum(z * scale[...] + shift[...], 0.0)
  h_out[...] = h
  q_out[...] = jax.lax.dot(h, wl1[...], preferred_element_type=jnp.float32)


def _dense0(s0p, degp, x, wl0, bl0, wr0, scale, shift, wl1):
  grid = (NPAD // BR,)
  return pl.pallas_call(
      _dense0_body,
      grid=grid,
      in_specs=[
          pl.BlockSpec((NC, BR, NFEAT), lambda i: (0, i, 0)),
          pl.BlockSpec((NC, BR, 16), lambda i: (0, i, 0)),
          pl.BlockSpec((BR, NFEAT), lambda i: (i, 0)),
          pl.BlockSpec((NFEAT, NHID), lambda i: (0, 0)),
          pl.BlockSpec((1, NHID), lambda i: (0, 0)),
          pl.BlockSpec((NFEAT, NHID), lambda i: (0, 0)),
          pl.BlockSpec((1, NHID), lambda i: (0, 0)),
          pl.BlockSpec((1, NHID), lambda i: (0, 0)),
          pl.BlockSpec((NHID, CPAD), lambda i: (0, 0)),
      ],
      out_specs=[
          pl.BlockSpec((BR, NHID), lambda i: (i, 0)),
          pl.BlockSpec((BR, CPAD), lambda i: (i, 0)),
      ],
      out_shape=[
          jax.ShapeDtypeStruct((NPAD, NHID), jnp.float32),
          jax.ShapeDtypeStruct((NPAD, CPAD), jnp.float32),
      ],
  )(s0p, degp, x, wl0, bl0, wr0, scale, shift, wl1)


def _dense1_body(s1p, degp, h, wr1, bl1, out):
  ssum = s1p[0] + s1p[1]
  deg = degp[0, :, 0:1] + degp[1, :, 0:1]
  z = (ssum / jnp.maximum(deg, 1.0) + bl1[...]
       + jax.lax.dot(h[...], wr1[...], preferred_element_type=jnp.float32))
  mask = lax.broadcasted_iota(jnp.int32, (1, CPAD), 1) < NCLASS
  z = jnp.where(mask, z, -1e30)
  m = jnp.max(z, axis=1, keepdims=True)
  ez = jnp.exp(z - m)
  lse = jnp.log(jnp.sum(ez, axis=1, keepdims=True))
  out[...] = z - m - lse


def _dense1(s1p, degp, h, wr1, bl1):
  grid = (NPAD // BR,)
  return pl.pallas_call(
      _dense1_body,
      grid=grid,
      in_specs=[
          pl.BlockSpec((NC, BR, CPAD), lambda i: (0, i, 0)),
          pl.BlockSpec((NC, BR, 16), lambda i: (0, i, 0)),
          pl.BlockSpec((BR, NHID), lambda i: (i, 0)),
          pl.BlockSpec((NHID, CPAD), lambda i: (0, 0)),
          pl.BlockSpec((1, CPAD), lambda i: (0, 0)),
      ],
      out_specs=pl.BlockSpec((BR, CPAD), lambda i: (i, 0)),
      out_shape=jax.ShapeDtypeStruct((NPAD, CPAD), jnp.float32),
  )(s1p, degp, h, wr1, bl1)


def kernel(x, edge_index, W_l0, b_l0, W_r0, gamma0, beta0, W_l1, b_l1, W_r1):
  src = edge_index[0]
  dst = edge_index[1]
  s0p, degp = _segsum_feat(x, src, dst)

  scale = (gamma0 / jnp.sqrt(1.0 + BN_EPS)).reshape(1, NHID)
  shift = beta0.reshape(1, NHID)
  wl1 = jnp.pad(W_l1, ((0, 0), (0, CPAD - NCLASS)))
  xpad = jnp.pad(x, ((0, NPAD - N), (0, 0)))
  h, q = _dense0(s0p, degp, xpad, W_l0, b_l0.reshape(1, NHID), W_r0,
                 scale, shift, wl1)

  (s1p,) = _segsum_cls(q, src, dst)

  wr1 = jnp.pad(W_r1, ((0, 0), (0, CPAD - NCLASS)))
  bl1 = jnp.pad(b_l1, (0, CPAD - NCLASS)).reshape(1, CPAD)
  out = _dense1(s1p, degp, h, wr1, bl1)
  return out[:N, :NCLASS]


# speedup vs baseline: 14.3752x; 2.6555x over previous
import functools

import jax
import jax.numpy as jnp
from jax import lax
from jax.experimental import pallas as pl
from jax.experimental.pallas import tpu as pltpu
from jax.experimental.pallas import tpu_sc as plsc

N = 10000
NPAD = 10240
E = 320000
NFEAT = 128
NHID = 128
NCLASS = 47
CPAD = 48
BN_EPS = 1e-5

NC = 2
NS = 16
NW = NC * NS
K = 80
RPT = NPAD // NS
ZB = 40
NB = 5
PF = 2

CPT_A = E // NS // K
CPT_B = E // NW // K


def _fill(ref, rows, width, value):
  v = jnp.full((16,), value, ref.dtype)
  for r in range(rows):
    for j in range(width // 16):
      ref[r, pl.ds(j * 16, 16)] = v


def _segsum_feat_kernel():
  mesh = plsc.VectorSubcoreMesh(core_axis_name="c", subcore_axis_name="s")
  out_type = (jax.ShapeDtypeStruct((NPAD, NFEAT), jnp.float32),
              jax.ShapeDtypeStruct((NPAD, 16), jnp.float32))
  scratch = [
      pltpu.VMEM((CPT_A, K), jnp.int32),
      pltpu.VMEM((CPT_A, K), jnp.int32),
      pltpu.VMEM((NB, K, 64), jnp.float32),
      pltpu.VMEM((ZB, 64), jnp.float32),
      pltpu.VMEM((K, 16), jnp.float32),
      pltpu.VMEM((ZB, 16), jnp.float32),
      pltpu.VMEM_SHARED((NPAD, 64), jnp.float32),
      pltpu.VMEM_SHARED((NPAD, 16), jnp.float32),
  ] + [pltpu.SemaphoreType.DMA] * (2 * NB + 1)

  def body(feat2, src, dst, out, deg_out, sidx, didx, rows, zbuf, ones,
           dzbuf, acc, dacc, *sems):
    gsem = sems[:NB]
    ssem = sems[NB:2 * NB]
    dsem = sems[2 * NB]
    c = lax.axis_index("c")
    s = lax.axis_index("s")
    on_c0 = c == 0

    _fill(zbuf, ZB, 64, 0.0)
    _fill(ones, K, 16, 1.0)
    _fill(dzbuf, ZB, 16, 0.0)

    pltpu.sync_copy(src.at[pl.ds(s * CPT_A, CPT_A)], sidx)
    pltpu.sync_copy(dst.at[pl.ds(s * CPT_A, CPT_A)], didx)

    def xform(r, _):
      for j in range(K // 16):
        sl = pl.ds(j * 16, 16)
        sidx[r, sl] = sidx[r, sl] * 2 + c
      return 0

    lax.fori_loop(0, CPT_A, xform, 0)

    r0 = s * RPT

    def zero_loop(i, _):
      pltpu.sync_copy(zbuf, acc.at[pl.ds(r0 + i * ZB, ZB)])

      @pl.when(on_c0)
      def _():
        pltpu.sync_copy(dzbuf, dacc.at[pl.ds(r0 + i * ZB, ZB)])
      return 0

    lax.fori_loop(0, RPT // ZB, zero_loop, 0)
    plsc.subcore_barrier()

    for b in range(PF):
      pltpu.async_copy(feat2.at[sidx.at[b]], rows.at[b], gsem[b])

    def outer(g, _):
      for b in range(NB):
        cs = g * NB + b
        bg = (b + PF) % NB

        @pl.when(jnp.logical_and(cs >= NB - PF, cs < CPT_A - PF))
        def _():
          pltpu.make_async_copy(rows.at[bg], acc.at[didx.at[0]],
                                ssem[bg]).wait()

        @pl.when(cs < CPT_A - PF)
        def _():
          pltpu.async_copy(feat2.at[sidx.at[cs + PF]], rows.at[bg], gsem[bg])

        pltpu.make_async_copy(feat2.at[sidx.at[cs]], rows.at[b],
                              gsem[b]).wait()
        pltpu.async_copy(rows.at[b], acc.at[didx.at[cs]], ssem[b], add=True)

        @pl.when(on_c0)
        def _():
          pltpu.async_copy(ones, dacc.at[didx.at[cs]], dsem, add=True)
      return 0

    lax.fori_loop(0, CPT_A // NB, outer, 0)

    for b in range(NB):
      pltpu.make_async_copy(rows.at[b], acc.at[didx.at[0]], ssem[b]).wait()

    @pl.when(on_c0)
    def _():
      def dloop(i, _):
        pltpu.make_async_copy(ones, dacc.at[didx.at[0]], dsem).wait()
        return 0
      lax.fori_loop(0, CPT_A, dloop, 0)

    plsc.subcore_barrier()

    pltpu.sync_copy(acc.at[pl.ds(r0, RPT)],
                    out.at[pl.ds(r0, RPT), pl.ds(c * 64, 64)])

    @pl.when(on_c0)
    def _():
      pltpu.sync_copy(dacc.at[pl.ds(r0, RPT)], deg_out.at[pl.ds(r0, RPT)])

  return pl.kernel(body, out_type=out_type, mesh=mesh,
                   scratch_types=tuple(scratch),
                   compiler_params=pltpu.CompilerParams(
                       use_tc_tiling_on_sc=False))


def _segsum_cls_kernel():
  mesh = plsc.VectorSubcoreMesh(core_axis_name="c", subcore_axis_name="s")
  out_type = jax.ShapeDtypeStruct((NC, NPAD, CPAD), jnp.float32)
  scratch = [
      pltpu.VMEM((CPT_B, K), jnp.int32),
      pltpu.VMEM((CPT_B, K), jnp.int32),
      pltpu.VMEM((NB, K, CPAD), jnp.float32),
      pltpu.VMEM((ZB, CPAD), jnp.float32),
      pltpu.VMEM_SHARED((NPAD, CPAD), jnp.float32),
  ] + [pltpu.SemaphoreType.DMA] * (2 * NB)

  def body(feat, src, dst, out, sidx, didx, rows, zbuf, acc, *sems):
    gsem = sems[:NB]
    ssem = sems[NB:2 * NB]
    c = lax.axis_index("c")
    s = lax.axis_index("s")
    w = c * NS + s

    _fill(zbuf, ZB, CPAD, 0.0)

    pltpu.sync_copy(src.at[pl.ds(w * CPT_B, CPT_B)], sidx)
    pltpu.sync_copy(dst.at[pl.ds(w * CPT_B, CPT_B)], didx)

    r0 = s * RPT

    def zero_loop(i, _):
      pltpu.sync_copy(zbuf, acc.at[pl.ds(r0 + i * ZB, ZB)])
      return 0

    lax.fori_loop(0, RPT // ZB, zero_loop, 0)
    plsc.subcore_barrier()

    for b in range(PF):
      pltpu.async_copy(feat.at[sidx.at[b]], rows.at[b], gsem[b])

    def outer(g, _):
      for b in range(NB):
        cs = g * NB + b
        bg = (b + PF) % NB

        @pl.when(jnp.logical_and(cs >= NB - PF, cs < CPT_B - PF))
        def _():
          pltpu.make_async_copy(rows.at[bg], acc.at[didx.at[0]],
                                ssem[bg]).wait()

        @pl.when(cs < CPT_B - PF)
        def _():
          pltpu.async_copy(feat.at[sidx.at[cs + PF]], rows.at[bg], gsem[bg])

        pltpu.make_async_copy(feat.at[sidx.at[cs]], rows.at[b],
                              gsem[b]).wait()
        pltpu.async_copy(rows.at[b], acc.at[didx.at[cs]], ssem[b], add=True)
      return 0

    lax.fori_loop(0, CPT_B // NB, outer, 0)

    for b in range(NB):
      pltpu.make_async_copy(rows.at[b], acc.at[didx.at[0]], ssem[b]).wait()
    plsc.subcore_barrier()

    pltpu.sync_copy(acc.at[pl.ds(r0, RPT)], out.at[c, pl.ds(r0, RPT)])

  return pl.kernel(body, out_type=out_type, mesh=mesh,
                   scratch_types=tuple(scratch),
                   compiler_params=pltpu.CompilerParams(
                       use_tc_tiling_on_sc=False))


_segsum_feat = _segsum_feat_kernel()
_segsum_cls = _segsum_cls_kernel()

BR = 1024


def _dense0_body(s0, degp, x, wl0, bl0, wr0, scale, shift, wl1, h_out, q_out):
  deg = degp[:, 0:1]
  mean = s0[...] / jnp.maximum(deg, 1.0)
  z = (jax.lax.dot(mean, wl0[...], preferred_element_type=jnp.float32)
       + bl0[...]
       + jax.lax.dot(x[...], wr0[...], preferred_element_type=jnp.float32))
  h = jnp.maximum(z * scale[...] + shift[...], 0.0)
  h_out[...] = h
  q_out[...] = jax.lax.dot(h, wl1[...], preferred_element_type=jnp.float32)


def _dense0(s0, degp, x, wl0, bl0, wr0, scale, shift, wl1):
  grid = (NPAD // BR,)
  return pl.pallas_call(
      _dense0_body,
      grid=grid,
      in_specs=[
          pl.BlockSpec((BR, NFEAT), lambda i: (i, 0)),
          pl.BlockSpec((BR, 16), lambda i: (i, 0)),
          pl.BlockSpec((BR, NFEAT), lambda i: (i, 0)),
          pl.BlockSpec((NFEAT, NHID), lambda i: (0, 0)),
          pl.BlockSpec((1, NHID), lambda i: (0, 0)),
          pl.BlockSpec((NFEAT, NHID), lambda i: (0, 0)),
          pl.BlockSpec((1, NHID), lambda i: (0, 0)),
          pl.BlockSpec((1, NHID), lambda i: (0, 0)),
          pl.BlockSpec((NHID, CPAD), lambda i: (0, 0)),
      ],
      out_specs=[
          pl.BlockSpec((BR, NHID), lambda i: (i, 0)),
          pl.BlockSpec((BR, CPAD), lambda i: (i, 0)),
      ],
      out_shape=[
          jax.ShapeDtypeStruct((NPAD, NHID), jnp.float32),
          jax.ShapeDtypeStruct((NPAD, CPAD), jnp.float32),
      ],
  )(s0, degp, x, wl0, bl0, wr0, scale, shift, wl1)


def _dense1_body(s1p, degp, h, wr1, bl1, out):
  ssum = s1p[0] + s1p[1]
  deg = degp[:, 0:1]
  z = (ssum / jnp.maximum(deg, 1.0) + bl1[...]
       + jax.lax.dot(h[...], wr1[...], preferred_element_type=jnp.float32))
  mask = lax.broadcasted_iota(jnp.int32, (1, CPAD), 1) < NCLASS
  z = jnp.where(mask, z, -1e30)
  m = jnp.max(z, axis=1, keepdims=True)
  ez = jnp.exp(z - m)
  lse = jnp.log(jnp.sum(ez, axis=1, keepdims=True))
  out[...] = z - m - lse


def _dense1(s1p, degp, h, wr1, bl1):
  grid = (NPAD // BR,)
  return pl.pallas_call(
      _dense1_body,
      grid=grid,
      in_specs=[
          pl.BlockSpec((NC, BR, CPAD), lambda i: (0, i, 0)),
          pl.BlockSpec((BR, 16), lambda i: (i, 0)),
          pl.BlockSpec((BR, NHID), lambda i: (i, 0)),
          pl.BlockSpec((NHID, CPAD), lambda i: (0, 0)),
          pl.BlockSpec((1, CPAD), lambda i: (0, 0)),
      ],
      out_specs=pl.BlockSpec((BR, CPAD), lambda i: (i, 0)),
      out_shape=jax.ShapeDtypeStruct((NPAD, CPAD), jnp.float32),
  )(s1p, degp, h, wr1, bl1)


def kernel(x, edge_index, W_l0, b_l0, W_r0, gamma0, beta0, W_l1, b_l1, W_r1):
  src = edge_index[0].reshape(E // K, K)
  dst = edge_index[1].reshape(E // K, K)
  x2 = x.reshape(2 * N, 64)
  s0, degp = _segsum_feat(x2, src, dst)

  scale = (gamma0 / jnp.sqrt(1.0 + BN_EPS)).reshape(1, NHID)
  shift = beta0.reshape(1, NHID)
  wl1 = jnp.pad(W_l1, ((0, 0), (0, CPAD - NCLASS)))
  xpad = jnp.pad(x, ((0, NPAD - N), (0, 0)))
  h, q = _dense0(s0, degp, xpad, W_l0, b_l0.reshape(1, NHID), W_r0,
                 scale, shift, wl1)

  s1p = _segsum_cls(q, src, dst)

  wr1 = jnp.pad(W_r1, ((0, 0), (0, CPAD - NCLASS)))
  bl1 = jnp.pad(b_l1, (0, CPAD - NCLASS)).reshape(1, CPAD)
  out = _dense1(s1p, degp, h, wr1, bl1)
  return out[:N, :NCLASS]
